# baseline (speedup 1.0000x reference)
import functools

import jax
import jax.numpy as jnp
from jax import lax
from jax.experimental import pallas as pl
from jax.experimental.pallas import tpu as pltpu
from jax.experimental.pallas import tpu_sc as plsc

MASK_PROB_C = 0.15
MASK_ID_C = 4
B_C, L_C = 1024, 2048
NUM_CORES = 2
NUM_SUBCORES = 16
NUM_WORKERS = NUM_CORES * NUM_SUBCORES
ROWS_PER_WORKER = B_C // NUM_WORKERS
LANES = 16
CHUNKS = L_C // LANES
G = 4
NG = ROWS_PER_WORKER // G
TAIL_W = float(L_C) + 3.0 + 1.0
NEG_SENT = -1e30
POS_SENT = 1e30
WPAD = L_C + 32


def _body(x_hbm, um_hbm, us_hbm, out_hbm,
          xa, uma, usa, oa, xb, umb, usb, ob, w_v,
          ina_sem, inb_sem, outa_sem, outb_sem):
    wid = lax.axis_index("s") * NUM_CORES + lax.axis_index("c")
    row0 = wid * ROWS_PER_WORKER
    lane = lax.broadcasted_iota(jnp.int32, (LANES,), 0)

    bufs = ((xa, uma, usa, oa, ina_sem, outa_sem),
            (xb, umb, usb, ob, inb_sem, outb_sem))

    def in_copies(g, bufset):
        bx, bum, bus, _, isem, _ = bufset
        rows = pl.ds(row0 + g * G, G)
        return (pltpu.make_async_copy(x_hbm.at[rows], bx, isem),
                pltpu.make_async_copy(um_hbm.at[rows], bum, isem),
                pltpu.make_async_copy(us_hbm.at[rows], bus, isem))

    def out_copy(g, bufset):
        bo, osem = bufset[3], bufset[5]
        rows = pl.ds(row0 + g * G, G)
        return pltpu.make_async_copy(bo, out_hbm.at[rows], osem)

    def do_group(bufset):
        bx, bum, bus, bo = bufset[:4]

        def do_row(j, _):
            w_v[pl.ds(0, LANES)] = jnp.full((LANES,), NEG_SENT, jnp.float32)
            w_v[pl.ds(L_C + 2, LANES)] = jnp.full((LANES,), POS_SENT, jnp.float32)

            @plsc.parallel_loop(0, CHUNKS, unroll=4)
            def pass1(i):
                b = i * LANES
                xv = bx[j, pl.ds(b, LANES)]
                us = bus[j, pl.ds(b, LANES)]
                idx = lane + b
                w = idx.astype(jnp.float32) + us * 3.0
                w = jnp.where(xv < 3, TAIL_W, w)
                w = jnp.where(idx == 0, -1.0, w)
                w_v[pl.ds(b + 2, LANES)] = w

            @plsc.parallel_loop(0, CHUNKS, unroll=4)
            def pass2(i):
                b = i * LANES
                w = w_v[pl.ds(b + 2, LANES)]
                wm2 = w_v[pl.ds(b, LANES)]
                wm1 = w_v[pl.ds(b + 1, LANES)]
                wp1 = w_v[pl.ds(b + 3, LANES)]
                wp2 = w_v[pl.ds(b + 4, LANES)]
                d = (
                    jnp.where(wp1 < w, 1, 0)
                    + jnp.where(wp2 < w, 1, 0)
                    - jnp.where(wm1 > w, 1, 0)
                    - jnp.where(wm2 > w, 1, 0)
                )
                xv = bx[j, pl.ds(b, LANES)]
                um = bum[j, pl.ds(b, LANES)]
                xm = jnp.where(um < MASK_PROB_C,
                               jnp.where(xv < 3, xv, MASK_ID_C), xv)
                row_idx = jnp.full((LANES,), j, jnp.int32)
                plsc.store_scatter(bo, [row_idx, lane + (b + d)], xm)

            return _

        lax.fori_loop(0, G, do_row, None)

    for c in in_copies(0, bufs[0]):
        c.start()
    for c in in_copies(1, bufs[1]):
        c.start()

    for g in range(NG):
        bufset = bufs[g % 2]
        for c in in_copies(g, bufset):
            c.wait()
        if g >= 2:
            out_copy(g - 2, bufset).wait()
        do_group(bufset)
        out_copy(g, bufset).start()
        if g + 2 < NG:
            for c in in_copies(g + 2, bufset):
                c.start()

    out_copy(NG - 2, bufs[NG % 2]).wait()
    out_copy(NG - 1, bufs[(NG + 1) % 2]).wait()


@jax.jit
def kernel(x, x_len, pad_sos_eos_mask, u_mask, u_shuffle):
    del x_len, pad_sos_eos_mask
    run = functools.partial(
        pl.kernel,
        out_type=jax.ShapeDtypeStruct((B_C, L_C), jnp.int32),
        mesh=plsc.VectorSubcoreMesh(core_axis_name="c", subcore_axis_name="s"),
        compiler_params=pltpu.CompilerParams(
            needs_layout_passes=False, skip_device_barrier=True
        ),
        scratch_types=[
            pltpu.VMEM((G, L_C), jnp.int32),
            pltpu.VMEM((G, L_C), jnp.float32),
            pltpu.VMEM((G, L_C), jnp.float32),
            pltpu.VMEM((G, L_C), jnp.int32),
            pltpu.VMEM((G, L_C), jnp.int32),
            pltpu.VMEM((G, L_C), jnp.float32),
            pltpu.VMEM((G, L_C), jnp.float32),
            pltpu.VMEM((G, L_C), jnp.int32),
            pltpu.VMEM((WPAD,), jnp.float32),
            pltpu.SemaphoreType.DMA,
            pltpu.SemaphoreType.DMA,
            pltpu.SemaphoreType.DMA,
            pltpu.SemaphoreType.DMA,
        ],
    )(_body)
    return run(x, u_mask, u_shuffle)


# speedup vs baseline: 1.0409x; 1.0409x over previous
import functools

import jax
import jax.numpy as jnp
from jax import lax
from jax.experimental import pallas as pl
from jax.experimental.pallas import tpu as pltpu
from jax.experimental.pallas import tpu_sc as plsc

MASK_PROB_C = 0.15
MASK_ID_C = 4
B_C, L_C = 1024, 2048
NUM_CORES = 2
NUM_SUBCORES = 16
NUM_WORKERS = NUM_CORES * NUM_SUBCORES
ROWS_PER_WORKER = B_C // NUM_WORKERS
LANES = 16
CHUNKS = L_C // LANES
G = 4
NG = ROWS_PER_WORKER // G
TAIL_W = float(L_C) + 3.0 + 1.0
NEG_SENT = -1e30
POS_SENT = 1e30
WPAD = L_C + 32


def _body(x_hbm, um_hbm, us_hbm, out_hbm,
          xa, uma, usa, oa, xb, umb, usb, ob, w_v,
          ina_sem, inb_sem, outa_sem, outb_sem):
    wid = lax.axis_index("s") * NUM_CORES + lax.axis_index("c")
    row0 = wid * ROWS_PER_WORKER
    lane = lax.broadcasted_iota(jnp.int32, (LANES,), 0)

    bufs = ((xa, uma, usa, oa, ina_sem, outa_sem),
            (xb, umb, usb, ob, inb_sem, outb_sem))

    def in_copies(g, bufset):
        bx, bum, bus, _, isem, _ = bufset
        rows = pl.ds(row0 + g * G, G)
        return (pltpu.make_async_copy(x_hbm.at[rows], bx, isem),
                pltpu.make_async_copy(um_hbm.at[rows], bum, isem),
                pltpu.make_async_copy(us_hbm.at[rows], bus, isem))

    def out_copy(g, bufset):
        bo, osem = bufset[3], bufset[5]
        rows = pl.ds(row0 + g * G, G)
        return pltpu.make_async_copy(bo, out_hbm.at[rows], osem)

    def do_group(bufset):
        bx, bum, bus, bo = bufset[:4]

        def do_row(j, _):
            w_v[pl.ds(0, LANES)] = jnp.full((LANES,), NEG_SENT, jnp.float32)
            w_v[pl.ds(L_C + 2, LANES)] = jnp.full((LANES,), POS_SENT, jnp.float32)

            @plsc.parallel_loop(0, CHUNKS, unroll=4)
            def pass1(i):
                b = i * LANES
                xv = bx[j, pl.ds(b, LANES)]
                us = bus[j, pl.ds(b, LANES)]
                idx = lane + b
                w = idx.astype(jnp.float32) + us * 3.0
                w = jnp.where(xv < 3, TAIL_W, w)
                w = jnp.where(idx == 0, -1.0, w)
                w_v[pl.ds(b + 2, LANES)] = w

            @plsc.parallel_loop(0, CHUNKS, unroll=4)
            def pass2(i):
                b = i * LANES
                w = w_v[pl.ds(b + 2, LANES)]
                wm2 = w_v[pl.ds(b, LANES)]
                wm1 = w_v[pl.ds(b + 1, LANES)]
                wp1 = w_v[pl.ds(b + 3, LANES)]
                wp2 = w_v[pl.ds(b + 4, LANES)]
                d = (
                    jnp.where(wp1 < w, 1, 0)
                    + jnp.where(wp2 < w, 1, 0)
                    - jnp.where(wm1 > w, 1, 0)
                    - jnp.where(wm2 > w, 1, 0)
                )
                xv = bx[j, pl.ds(b, LANES)]
                um = bum[j, pl.ds(b, LANES)]
                xm = jnp.where(um < MASK_PROB_C,
                               jnp.where(xv < 3, xv, MASK_ID_C), xv)
                row_idx = jnp.full((LANES,), j, jnp.int32)
                plsc.store_scatter(bo, [row_idx, lane + (b + d)], xm)

            return _

        lax.fori_loop(0, G, do_row, None)

    for c in in_copies(0, bufs[0]):
        c.start()
    for c in in_copies(1, bufs[1]):
        c.start()

    def pair(p, _):
        for parity in (0, 1):
            g = 2 * p + parity
            bufset = bufs[parity]
            for c in in_copies(g, bufset):
                c.wait()

            @pl.when(p > 0)
            def _wait_out():
                out_copy(g - 2, bufset).wait()

            do_group(bufset)
            out_copy(g, bufset).start()

            @pl.when(p < NG // 2 - 1)
            def _prefetch():
                for c in in_copies(g + 2, bufset):
                    c.start()

        return _

    lax.fori_loop(0, NG // 2, pair, None)

    out_copy(NG - 2, bufs[0]).wait()
    out_copy(NG - 1, bufs[1]).wait()


@jax.jit
def kernel(x, x_len, pad_sos_eos_mask, u_mask, u_shuffle):
    del x_len, pad_sos_eos_mask
    run = functools.partial(
        pl.kernel,
        out_type=jax.ShapeDtypeStruct((B_C, L_C), jnp.int32),
        mesh=plsc.VectorSubcoreMesh(core_axis_name="c", subcore_axis_name="s"),
        compiler_params=pltpu.CompilerParams(needs_layout_passes=False),
        scratch_types=[
            pltpu.VMEM((G, L_C), jnp.int32),
            pltpu.VMEM((G, L_C), jnp.float32),
            pltpu.VMEM((G, L_C), jnp.float32),
            pltpu.VMEM((G, L_C), jnp.int32),
            pltpu.VMEM((G, L_C), jnp.int32),
            pltpu.VMEM((G, L_C), jnp.float32),
            pltpu.VMEM((G, L_C), jnp.float32),
            pltpu.VMEM((G, L_C), jnp.int32),
            pltpu.VMEM((WPAD,), jnp.float32),
            pltpu.SemaphoreType.DMA,
            pltpu.SemaphoreType.DMA,
            pltpu.SemaphoreType.DMA,
            pltpu.SemaphoreType.DMA,
        ],
    )(_body)
    return run(x, u_mask, u_shuffle)
